# baseline (speedup 1.0000x reference)
import functools
import jax
import jax.numpy as jnp
from jax import lax
from jax.experimental import pallas as pl
from jax.experimental.pallas import tpu as pltpu
from jax.experimental.pallas import tpu_sc as plsc

B = 128
H = 2048
I = 1024
K = 8
LOCAL = 8
NSPLIT = 2
IS = I // NSPLIT

_SC_MESH = plsc.VectorSubcoreMesh(core_axis_name="c", subcore_axis_name="s")


@functools.partial(
    pl.kernel,
    out_type=jax.ShapeDtypeStruct((LOCAL, 1, B), jnp.float32),
    mesh=_SC_MESH,
    scratch_types=[
        pltpu.VMEM((B * K,), jnp.int32),
        pltpu.VMEM((B * K,), jnp.float32),
        pltpu.VMEM((B,), jnp.float32),
        pltpu.VMEM((B,), jnp.float32),
    ],
)
def _routing_sc(eid_hbm, sc_hbm, act_hbm, out_hbm, eid_v, sc_v, act_v, row_v):
    wid = lax.axis_index("s") * 2 + lax.axis_index("c")

    @pl.when(wid < LOCAL)
    def _():
        pltpu.sync_copy(eid_hbm, eid_v)
        pltpu.sync_copy(sc_hbm, sc_v)
        pltpu.sync_copy(act_hbm, act_v)
        e = wid
        for chunk in range(B // 16):
            acc = jnp.zeros((16,), jnp.float32)
            for k in range(K):
                off = k * B + chunk * 16
                ev = eid_v[pl.ds(off, 16)]
                sv = sc_v[pl.ds(off, 16)]
                acc = acc + jnp.where(ev == e, sv, 0.0)
            row_v[pl.ds(chunk * 16, 16)] = acc * act_v[pl.ds(chunk * 16, 16)]
        pltpu.sync_copy(row_v, out_hbm.at[e, 0])


def _moe_body(x_ref, w1g_ref, w1u_ref, w2_ref, w_ref, out_ref):
    e = pl.program_id(0)
    j = pl.program_id(1)
    f32 = jnp.float32
    bf16 = jnp.bfloat16
    xb = x_ref[...].astype(bf16)
    gate = jnp.dot(xb, w1g_ref[0].astype(bf16), preferred_element_type=f32)
    up = jnp.dot(xb, w1u_ref[0].astype(bf16), preferred_element_type=f32)
    w = w_ref[0, 0, :]
    a = gate * jax.nn.sigmoid(gate) * up * w[:, None]

    @pl.when(jnp.logical_and(e == 0, j == 0))
    def _():
        out_ref[...] = jnp.zeros_like(out_ref)

    out_ref[...] += jnp.dot(a.astype(bf16), w2_ref[0].astype(bf16),
                            preferred_element_type=f32)


def kernel(x, expert_ids, smooth_scales, expert_scales, x_active_mask,
           gmm1_weight, gmm2_weight):
    del smooth_scales
    eids = expert_ids.astype(jnp.int32).T.reshape(K * B)
    scf = expert_scales.T.reshape(K * B)
    actf = x_active_mask.astype(jnp.float32)
    w_route = _routing_sc(eids, scf, actf)

    out = pl.pallas_call(
        _moe_body,
        grid=(LOCAL, NSPLIT),
        in_specs=[
            pl.BlockSpec((B, H), lambda e, j: (0, 0)),
            pl.BlockSpec((1, H, IS), lambda e, j: (e, 0, j)),
            pl.BlockSpec((1, H, IS), lambda e, j: (e, 0, NSPLIT + j)),
            pl.BlockSpec((1, IS, H), lambda e, j: (e, j, 0)),
            pl.BlockSpec((1, 1, B), lambda e, j: (e, 0, 0)),
        ],
        out_specs=pl.BlockSpec((B, H), lambda e, j: (0, 0)),
        out_shape=jax.ShapeDtypeStruct((B, H), jnp.float32),
    )(x, gmm1_weight, gmm1_weight, gmm2_weight, w_route)
    return out


# speedup vs baseline: 1.0293x; 1.0293x over previous
import functools
import jax
import jax.numpy as jnp
from jax import lax
from jax.experimental import pallas as pl
from jax.experimental.pallas import tpu as pltpu
from jax.experimental.pallas import tpu_sc as plsc

B = 128
H = 2048
I = 1024
K = 8
LOCAL = 8

_SC_MESH = plsc.VectorSubcoreMesh(core_axis_name="c", subcore_axis_name="s")


@functools.partial(
    pl.kernel,
    out_type=jax.ShapeDtypeStruct((LOCAL, 1, B), jnp.float32),
    mesh=_SC_MESH,
    scratch_types=[
        pltpu.VMEM((K * B,), jnp.int32),
        pltpu.VMEM((K * B,), jnp.float32),
        pltpu.VMEM((B,), jnp.float32),
        pltpu.VMEM((B,), jnp.float32),
    ],
)
def _routing_sc(eid_hbm, sc_hbm, act_hbm, out_hbm, eid_v, sc_v, act_v, row_v):
    wid = lax.axis_index("s") * 2 + lax.axis_index("c")

    @pl.when(wid < LOCAL)
    def _():
        pltpu.sync_copy(eid_hbm, eid_v)
        pltpu.sync_copy(sc_hbm, sc_v)
        pltpu.sync_copy(act_hbm, act_v)
        e = wid
        for chunk in range(B // 16):
            acc = jnp.zeros((16,), jnp.float32)
            for k in range(K):
                off = k * B + chunk * 16
                ev = eid_v[pl.ds(off, 16)]
                sv = sc_v[pl.ds(off, 16)]
                acc = acc + jnp.where(ev == e, sv, 0.0)
            row_v[pl.ds(chunk * 16, 16)] = acc * act_v[pl.ds(chunk * 16, 16)]
        pltpu.sync_copy(row_v, out_hbm.at[e, 0])


def _mlp1_body(x_ref, w1_ref, act_ref):
    f32 = jnp.float32
    bf16 = jnp.bfloat16
    xb = x_ref[...].astype(bf16)
    h1 = jnp.dot(xb, w1_ref[0].astype(bf16), preferred_element_type=f32)
    gate = h1[:, :I]
    up = h1[:, I:]
    act_ref[0] = (gate * jax.nn.sigmoid(gate) * up).astype(bf16)


def _mlp2_body(act_ref, w2_ref, w_ref, out_ref):
    e = pl.program_id(0)

    @pl.when(e == 0)
    def _():
        out_ref[...] = jnp.zeros_like(out_ref)

    a = (act_ref[0].astype(jnp.float32)
         * w_ref[0, 0, :][:, None]).astype(jnp.bfloat16)
    out_ref[...] += jnp.dot(a, w2_ref[0].astype(jnp.bfloat16),
                            preferred_element_type=jnp.float32)


def kernel(x, expert_ids, smooth_scales, expert_scales, x_active_mask,
           gmm1_weight, gmm2_weight):
    del smooth_scales
    eids = expert_ids.astype(jnp.int32).T.reshape(K * B)
    scf = expert_scales.T.reshape(K * B)
    actf = x_active_mask.astype(jnp.float32)
    w_route = _routing_sc(eids, scf, actf)

    act = pl.pallas_call(
        _mlp1_body,
        grid=(LOCAL,),
        in_specs=[
            pl.BlockSpec((B, H), lambda e: (0, 0)),
            pl.BlockSpec((1, H, 2 * I), lambda e: (e, 0, 0)),
        ],
        out_specs=pl.BlockSpec((1, B, I), lambda e: (e, 0, 0)),
        out_shape=jax.ShapeDtypeStruct((LOCAL, B, I), jnp.bfloat16),
    )(x, gmm1_weight)

    out = pl.pallas_call(
        _mlp2_body,
        grid=(LOCAL,),
        in_specs=[
            pl.BlockSpec((1, B, I), lambda e: (e, 0, 0)),
            pl.BlockSpec((1, I, H), lambda e: (e, 0, 0)),
            pl.BlockSpec((1, 1, B), lambda e: (e, 0, 0)),
        ],
        out_specs=pl.BlockSpec((B, H), lambda e: (0, 0)),
        out_shape=jax.ShapeDtypeStruct((B, H), jnp.float32),
    )(act, gmm2_weight, w_route)
    return out
